# baseline (speedup 1.0000x reference)
import functools

import jax
import jax.numpy as jnp
from jax.experimental import pallas as pl

_EPS = 1e-5
_TILE = 512
_X3 = (True, True, True)

_bf16 = jnp.bfloat16
_f32 = jnp.float32


def _mm(a32, wh_ref, wl_ref=None):
    ah = a32.astype(_bf16)
    out = jnp.dot(ah, wh_ref[...], preferred_element_type=_f32)
    if wl_ref is not None:
        al = (a32 - ah.astype(_f32)).astype(_bf16)
        out = out + jnp.dot(ah, wl_ref[...], preferred_element_type=_f32)
        out = out + jnp.dot(al, wh_ref[...], preferred_element_type=_f32)
    return out


def _accum_stats(i, z, st_ref):
    @pl.when(i == 0)
    def _():
        st_ref[...] = jnp.zeros(st_ref.shape, _f32)

    st_ref[0:1, :] += jnp.sum(z, axis=0, keepdims=True)
    st_ref[1:2, :] += jnp.sum(z * z, axis=0, keepdims=True)


def _bn_coeffs(stin_ref, g_ref, be_ref, inv_n):
    mu = stin_ref[0:1, :] * inv_n
    var = stin_ref[1:2, :] * inv_n - mu * mu
    scale = g_ref[...] * jax.lax.rsqrt(var + _EPS)
    shift = be_ref[...] - mu * scale
    return scale, shift


def _k1(num_dem, src_ref, emb_ref, w1d_ref, w1e_ref, b1_ref, z_ref, st_ref):
    i = pl.program_id(0)
    src = src_ref[...]
    dem = src[:, :num_dem]
    codes = src[:, num_dem:]
    mh32 = (codes != 0.0).astype(_f32)
    counts = jnp.maximum(jnp.sum(mh32, axis=1, keepdims=True), 1.0)
    emb = jnp.dot(mh32.astype(_bf16), emb_ref[...], preferred_element_type=_f32)
    emb = emb / counts
    z = jnp.dot(emb.astype(_bf16), w1e_ref[...], preferred_element_type=_f32)
    z = z + dem[:, 0:1] * w1d_ref[0:1, :] + dem[:, 1:2] * w1d_ref[1:2, :]
    z = z + b1_ref[...]
    z_ref[...] = z
    _accum_stats(i, z, st_ref)


def _kmid_x3(inv_n, zin_ref, stin_ref, g_ref, be_ref, wh_ref, wl_ref, b_ref,
             z_ref, st_ref):
    i = pl.program_id(0)
    scale, shift = _bn_coeffs(stin_ref, g_ref, be_ref, inv_n)
    h = jnp.maximum(zin_ref[...] * scale + shift, 0.0)
    z = _mm(h, wh_ref, wl_ref) + b_ref[...]
    z_ref[...] = z
    _accum_stats(i, z, st_ref)


def _kmid_x1(inv_n, zin_ref, stin_ref, g_ref, be_ref, wh_ref, b_ref,
             z_ref, st_ref):
    i = pl.program_id(0)
    scale, shift = _bn_coeffs(stin_ref, g_ref, be_ref, inv_n)
    h = jnp.maximum(zin_ref[...] * scale + shift, 0.0)
    z = _mm(h, wh_ref) + b_ref[...]
    z_ref[...] = z
    _accum_stats(i, z, st_ref)


def _klast_x3(inv_n, zin_ref, stin_ref, g_ref, be_ref, wh_ref, wl_ref, b_ref,
              out_ref):
    scale, shift = _bn_coeffs(stin_ref, g_ref, be_ref, inv_n)
    h = jnp.maximum(zin_ref[...] * scale + shift, 0.0)
    out_ref[...] = _mm(h, wh_ref, wl_ref) + b_ref[...]


def _klast_x1(inv_n, zin_ref, stin_ref, g_ref, be_ref, wh_ref, b_ref, out_ref):
    scale, shift = _bn_coeffs(stin_ref, g_ref, be_ref, inv_n)
    h = jnp.maximum(zin_ref[...] * scale + shift, 0.0)
    out_ref[...] = _mm(h, wh_ref) + b_ref[...]


def _full(shape):
    return pl.BlockSpec(shape, lambda i: (0, 0))


def _split(w, x3):
    wh = w.astype(_bf16)
    if not x3:
        return (wh,)
    return wh, (w - wh.astype(_f32)).astype(_bf16)


def kernel(src, embed, W1, b1, g1, be1, W2, b2, g2, be2, W3, b3, g3, be3,
           W4, b4):
    batch, d_in = src.shape
    vocab, ed = embed.shape
    num_dem = d_in - vocab
    h1, h2, h3, nb = W1.shape[1], W2.shape[1], W3.shape[1], W4.shape[1]
    nblk = batch // _TILE
    inv_n = 1.0 / batch
    grid = (nblk,)

    def tiled(f):
        return pl.BlockSpec((_TILE, f), lambda i: (i, 0))

    def row(a):
        return a.reshape(1, -1)

    z1, st1 = pl.pallas_call(
        functools.partial(_k1, num_dem),
        grid=grid,
        in_specs=[tiled(d_in), _full((vocab, ed)), _full((num_dem, h1)),
                  _full((ed, h1)), _full((1, h1))],
        out_specs=[tiled(h1), _full((8, h1))],
        out_shape=[jax.ShapeDtypeStruct((batch, h1), _f32),
                   jax.ShapeDtypeStruct((8, h1), _f32)],
    )(src, embed.astype(_bf16), W1[:num_dem], W1[num_dem:].astype(_bf16),
      row(b1))

    def mid(zin, stin, g, be, w, b, fin, fout, x3):
        ws = _split(w, x3)
        body = _kmid_x3 if x3 else _kmid_x1
        return pl.pallas_call(
            functools.partial(body, inv_n),
            grid=grid,
            in_specs=[tiled(fin), _full((8, fin)), _full((1, fin)),
                      _full((1, fin))]
                     + [_full((fin, fout))] * len(ws) + [_full((1, fout))],
            out_specs=[tiled(fout), _full((8, fout))],
            out_shape=[jax.ShapeDtypeStruct((batch, fout), _f32),
                       jax.ShapeDtypeStruct((8, fout), _f32)],
        )(zin, stin, row(g), row(be), *ws, row(b))

    z2, st2 = mid(z1, st1, g1, be1, W2, b2, h1, h2, _X3[0])
    z3, st3 = mid(z2, st2, g2, be2, W3, b3, h2, h3, _X3[1])

    ws4 = _split(W4, _X3[2])
    body = _klast_x3 if _X3[2] else _klast_x1
    pred = pl.pallas_call(
        functools.partial(body, inv_n),
        grid=grid,
        in_specs=[tiled(h3), _full((8, h3)), _full((1, h3)), _full((1, h3))]
                 + [_full((h3, nb))] * len(ws4) + [_full((1, nb))],
        out_specs=tiled(nb),
        out_shape=jax.ShapeDtypeStruct((batch, nb), _f32),
    )(z3, st3, row(g3), row(be3), *ws4, row(b4))
    return pred


# speedup vs baseline: 1.2981x; 1.2981x over previous
import functools

import jax
import jax.numpy as jnp
from jax.experimental import pallas as pl

_EPS = 1e-5
_TILE = 512
_X3 = (False, False, False)

_bf16 = jnp.bfloat16
_f32 = jnp.float32


def _mm(a32, wh_ref, wl_ref=None):
    ah = a32.astype(_bf16)
    out = jnp.dot(ah, wh_ref[...], preferred_element_type=_f32)
    if wl_ref is not None:
        al = (a32 - ah.astype(_f32)).astype(_bf16)
        out = out + jnp.dot(ah, wl_ref[...], preferred_element_type=_f32)
        out = out + jnp.dot(al, wh_ref[...], preferred_element_type=_f32)
    return out


def _accum_stats(i, z, st_ref):
    @pl.when(i == 0)
    def _():
        st_ref[...] = jnp.zeros(st_ref.shape, _f32)

    st_ref[0:1, :] += jnp.sum(z, axis=0, keepdims=True)
    st_ref[1:2, :] += jnp.sum(z * z, axis=0, keepdims=True)


def _bn_coeffs(stin_ref, g_ref, be_ref, inv_n):
    mu = stin_ref[0:1, :] * inv_n
    var = stin_ref[1:2, :] * inv_n - mu * mu
    scale = g_ref[...] * jax.lax.rsqrt(var + _EPS)
    shift = be_ref[...] - mu * scale
    return scale, shift


def _k1(num_dem, src_ref, emb_ref, w1d_ref, w1e_ref, b1_ref, z_ref, st_ref):
    i = pl.program_id(0)
    src = src_ref[...]
    dem = src[:, :num_dem]
    codes = src[:, num_dem:]
    mh32 = (codes != 0.0).astype(_f32)
    counts = jnp.maximum(jnp.sum(mh32, axis=1, keepdims=True), 1.0)
    emb = jnp.dot(mh32.astype(_bf16), emb_ref[...], preferred_element_type=_f32)
    emb = emb / counts
    z = jnp.dot(emb.astype(_bf16), w1e_ref[...], preferred_element_type=_f32)
    z = z + dem[:, 0:1] * w1d_ref[0:1, :] + dem[:, 1:2] * w1d_ref[1:2, :]
    z = z + b1_ref[...]
    z_ref[...] = z
    _accum_stats(i, z, st_ref)


def _kmid_x3(inv_n, zin_ref, stin_ref, g_ref, be_ref, wh_ref, wl_ref, b_ref,
             z_ref, st_ref):
    i = pl.program_id(0)
    scale, shift = _bn_coeffs(stin_ref, g_ref, be_ref, inv_n)
    h = jnp.maximum(zin_ref[...] * scale + shift, 0.0)
    z = _mm(h, wh_ref, wl_ref) + b_ref[...]
    z_ref[...] = z
    _accum_stats(i, z, st_ref)


def _kmid_x1(inv_n, zin_ref, stin_ref, g_ref, be_ref, wh_ref, b_ref,
             z_ref, st_ref):
    i = pl.program_id(0)
    scale, shift = _bn_coeffs(stin_ref, g_ref, be_ref, inv_n)
    h = jnp.maximum(zin_ref[...] * scale + shift, 0.0)
    z = _mm(h, wh_ref) + b_ref[...]
    z_ref[...] = z
    _accum_stats(i, z, st_ref)


def _klast_x3(inv_n, zin_ref, stin_ref, g_ref, be_ref, wh_ref, wl_ref, b_ref,
              out_ref):
    scale, shift = _bn_coeffs(stin_ref, g_ref, be_ref, inv_n)
    h = jnp.maximum(zin_ref[...] * scale + shift, 0.0)
    out_ref[...] = _mm(h, wh_ref, wl_ref) + b_ref[...]


def _klast_x1(inv_n, zin_ref, stin_ref, g_ref, be_ref, wh_ref, b_ref, out_ref):
    scale, shift = _bn_coeffs(stin_ref, g_ref, be_ref, inv_n)
    h = jnp.maximum(zin_ref[...] * scale + shift, 0.0)
    out_ref[...] = _mm(h, wh_ref) + b_ref[...]


def _full(shape):
    return pl.BlockSpec(shape, lambda i: (0, 0))


def _split(w, x3):
    wh = w.astype(_bf16)
    if not x3:
        return (wh,)
    return wh, (w - wh.astype(_f32)).astype(_bf16)


def kernel(src, embed, W1, b1, g1, be1, W2, b2, g2, be2, W3, b3, g3, be3,
           W4, b4):
    batch, d_in = src.shape
    vocab, ed = embed.shape
    num_dem = d_in - vocab
    h1, h2, h3, nb = W1.shape[1], W2.shape[1], W3.shape[1], W4.shape[1]
    nblk = batch // _TILE
    inv_n = 1.0 / batch
    grid = (nblk,)

    def tiled(f):
        return pl.BlockSpec((_TILE, f), lambda i: (i, 0))

    def row(a):
        return a.reshape(1, -1)

    z1, st1 = pl.pallas_call(
        functools.partial(_k1, num_dem),
        grid=grid,
        in_specs=[tiled(d_in), _full((vocab, ed)), _full((num_dem, h1)),
                  _full((ed, h1)), _full((1, h1))],
        out_specs=[tiled(h1), _full((8, h1))],
        out_shape=[jax.ShapeDtypeStruct((batch, h1), _f32),
                   jax.ShapeDtypeStruct((8, h1), _f32)],
    )(src, embed.astype(_bf16), W1[:num_dem], W1[num_dem:].astype(_bf16),
      row(b1))

    def mid(zin, stin, g, be, w, b, fin, fout, x3):
        ws = _split(w, x3)
        body = _kmid_x3 if x3 else _kmid_x1
        return pl.pallas_call(
            functools.partial(body, inv_n),
            grid=grid,
            in_specs=[tiled(fin), _full((8, fin)), _full((1, fin)),
                      _full((1, fin))]
                     + [_full((fin, fout))] * len(ws) + [_full((1, fout))],
            out_specs=[tiled(fout), _full((8, fout))],
            out_shape=[jax.ShapeDtypeStruct((batch, fout), _f32),
                       jax.ShapeDtypeStruct((8, fout), _f32)],
        )(zin, stin, row(g), row(be), *ws, row(b))

    z2, st2 = mid(z1, st1, g1, be1, W2, b2, h1, h2, _X3[0])
    z3, st3 = mid(z2, st2, g2, be2, W3, b3, h2, h3, _X3[1])

    ws4 = _split(W4, _X3[2])
    body = _klast_x3 if _X3[2] else _klast_x1
    pred = pl.pallas_call(
        functools.partial(body, inv_n),
        grid=grid,
        in_specs=[tiled(h3), _full((8, h3)), _full((1, h3)), _full((1, h3))]
                 + [_full((h3, nb))] * len(ws4) + [_full((1, nb))],
        out_specs=tiled(nb),
        out_shape=jax.ShapeDtypeStruct((batch, nb), _f32),
    )(z3, st3, row(g3), row(be3), *ws4, row(b4))
    return pred


# speedup vs baseline: 1.4644x; 1.1281x over previous
import functools

import jax
import jax.numpy as jnp
from jax.experimental import pallas as pl
from jax.experimental.pallas import tpu as pltpu

_EPS = 1e-5
_TILE = 512

_bf16 = jnp.bfloat16
_f32 = jnp.float32


def _accum_stats(i, z, st_ref):
    @pl.when(i == 0)
    def _():
        st_ref[...] = jnp.zeros(st_ref.shape, _f32)

    st_ref[0:1, :] += jnp.sum(z, axis=0, keepdims=True)
    st_ref[1:2, :] += jnp.sum(z * z, axis=0, keepdims=True)


def _bn_coeffs(stin_ref, g_ref, be_ref, inv_n):
    mu = stin_ref[0:1, :] * inv_n
    var = stin_ref[1:2, :] * inv_n - mu * mu
    scale = g_ref[...] * jax.lax.rsqrt(var + _EPS)
    shift = be_ref[...] - mu * scale
    return scale, shift


def _k1(num_dem, ed, src_ref, emb_ref, w1d_ref, w1e_ref, b1_ref, z_ref,
        st_ref):
    i = pl.program_id(0)
    src = src_ref[...]
    dem = src[:, :num_dem]
    codes = src[:, num_dem:]
    mh = (codes != 0.0).astype(_bf16)
    pooled = jnp.dot(mh, emb_ref[...], preferred_element_type=_f32)
    counts = jnp.maximum(pooled[:, ed:ed + 1], 1.0)
    emb = pooled[:, :ed] * (1.0 / counts)
    z = jnp.dot(emb.astype(_bf16), w1e_ref[...], preferred_element_type=_f32)
    z = z + dem[:, 0:1] * w1d_ref[0:1, :] + dem[:, 1:2] * w1d_ref[1:2, :]
    z = z + b1_ref[...]
    z_ref[...] = z
    _accum_stats(i, z, st_ref)


def _kmid(inv_n, zin_ref, stin_ref, g_ref, be_ref, w_ref, b_ref,
          z_ref, st_ref, wbf_ref):
    i = pl.program_id(0)

    @pl.when(i == 0)
    def _():
        wbf_ref[...] = w_ref[...].astype(_bf16)

    scale, shift = _bn_coeffs(stin_ref, g_ref, be_ref, inv_n)
    h = jnp.maximum(zin_ref[...].astype(_f32) * scale + shift, 0.0)
    z = jnp.dot(h.astype(_bf16), wbf_ref[...],
                preferred_element_type=_f32) + b_ref[...]
    z_ref[...] = z.astype(z_ref.dtype)
    _accum_stats(i, z, st_ref)


def _klast(inv_n, zin_ref, stin_ref, g_ref, be_ref, w_ref, b_ref, out_ref,
           wbf_ref):
    i = pl.program_id(0)

    @pl.when(i == 0)
    def _():
        wbf_ref[...] = w_ref[...].astype(_bf16)

    scale, shift = _bn_coeffs(stin_ref, g_ref, be_ref, inv_n)
    h = jnp.maximum(zin_ref[...].astype(_f32) * scale + shift, 0.0)
    out_ref[...] = jnp.dot(h.astype(_bf16), wbf_ref[...],
                           preferred_element_type=_f32) + b_ref[...]


def _full(shape):
    return pl.BlockSpec(shape, lambda i: (0, 0))


def kernel(src, embed, W1, b1, g1, be1, W2, b2, g2, be2, W3, b3, g3, be3,
           W4, b4):
    batch, d_in = src.shape
    vocab, ed = embed.shape
    num_dem = d_in - vocab
    h1, h2, h3, nb = W1.shape[1], W2.shape[1], W3.shape[1], W4.shape[1]
    nblk = batch // _TILE
    inv_n = 1.0 / batch
    grid = (nblk,)

    emb_aug = jnp.concatenate(
        [embed, jnp.ones((vocab, 1), _f32)], axis=1).astype(_bf16)

    def tiled(f, dt=_f32):
        del dt
        return pl.BlockSpec((_TILE, f), lambda i: (i, 0))

    def row(a):
        return a.reshape(1, -1)

    z1, st1 = pl.pallas_call(
        functools.partial(_k1, num_dem, ed),
        grid=grid,
        in_specs=[tiled(d_in), _full((vocab, ed + 1)), _full((num_dem, h1)),
                  _full((ed, h1)), _full((1, h1))],
        out_specs=[tiled(h1), _full((8, h1))],
        out_shape=[jax.ShapeDtypeStruct((batch, h1), _f32),
                   jax.ShapeDtypeStruct((8, h1), _f32)],
    )(src, emb_aug, W1[:num_dem], W1[num_dem:].astype(_bf16), row(b1))

    def mid(zin, stin, g, be, w, b, fin, fout):
        return pl.pallas_call(
            functools.partial(_kmid, inv_n),
            grid=grid,
            in_specs=[tiled(fin), _full((8, fin)), _full((1, fin)),
                      _full((1, fin)), _full((fin, fout)), _full((1, fout))],
            out_specs=[tiled(fout), _full((8, fout))],
            out_shape=[jax.ShapeDtypeStruct((batch, fout), _bf16),
                       jax.ShapeDtypeStruct((8, fout), _f32)],
            scratch_shapes=[pltpu.VMEM((fin, fout), _bf16)],
        )(zin, stin, row(g), row(be), w, row(b))

    z2, st2 = mid(z1, st1, g1, be1, W2, b2, h1, h2)
    z3, st3 = mid(z2, st2, g2, be2, W3, b3, h2, h3)

    pred = pl.pallas_call(
        functools.partial(_klast, inv_n),
        grid=grid,
        in_specs=[tiled(h3), _full((8, h3)), _full((1, h3)), _full((1, h3)),
                  _full((h3, nb)), _full((1, nb))],
        out_specs=tiled(nb),
        out_shape=jax.ShapeDtypeStruct((batch, nb), _f32),
        scratch_shapes=[pltpu.VMEM((h3, nb), _bf16)],
    )(z3, st3, row(g3), row(be3), W4, row(b4))
    return pred
